# Initial kernel scaffold; baseline (speedup 1.0000x reference)
import jax
import jax.numpy as jnp
from jax.experimental import pallas as pl


def kernel(x_gene_id, x_connect_id, x_rna_type, basic_table, homo_table, rna_table):
    raise NotImplementedError("write your pallas kernel here")


# speedup vs baseline: 2.7286x; 2.7286x over previous
import functools

import jax
import jax.numpy as jnp
from jax import lax
from jax.experimental import pallas as pl
from jax.experimental.pallas import tpu as pltpu
from jax.experimental.pallas import tpu_sc as plsc

DIM = 64
LANES = 16
CHUNK = 128


@functools.lru_cache(maxsize=None)
def _build(n_rows: int):
    info = plsc.get_sparse_core_info()
    num_workers = info.num_cores * info.num_subcores
    per_w = n_rows // num_workers
    n_chunks = per_w // CHUNK
    assert per_w * num_workers == n_rows and n_chunks * CHUNK == per_w

    mesh = plsc.VectorSubcoreMesh(core_axis_name="c", subcore_axis_name="s")

    @functools.partial(
        pl.kernel,
        mesh=mesh,
        compiler_params=pltpu.CompilerParams(use_tc_tiling_on_sc=False),
        out_type=jax.ShapeDtypeStruct((n_rows, DIM), jnp.float32),
        scratch_types=[
            pltpu.VMEM((CHUNK,), jnp.int32),
            pltpu.VMEM((CHUNK,), jnp.int32),
            pltpu.VMEM((CHUNK,), jnp.int32),
            pltpu.VMEM((CHUNK, DIM), jnp.float32),
            pltpu.VMEM((CHUNK, DIM), jnp.float32),
            pltpu.VMEM((CHUNK, DIM), jnp.float32),
            pltpu.SemaphoreType.DMA,
            pltpu.SemaphoreType.DMA,
            pltpu.SemaphoreType.DMA,
        ],
    )
    def emb_sum(gid, cid, rid, basic, homo, rna, out,
                gidx_v, cidx_v, ridx_v, acc_v, h_v, r_v, sem0, sem1, sem2):
        w = lax.axis_index("s") * info.num_cores + lax.axis_index("c")
        w_base = w * per_w

        def chunk_body(g, carry):
            base = w_base + g * CHUNK
            pltpu.sync_copy(gid.at[pl.ds(base, CHUNK)], gidx_v)
            pltpu.sync_copy(cid.at[pl.ds(base, CHUNK)], cidx_v)
            pltpu.sync_copy(rid.at[pl.ds(base, CHUNK)], ridx_v)
            ca = pltpu.async_copy(basic.at[gidx_v], acc_v, sem0)
            cb = pltpu.async_copy(homo.at[cidx_v], h_v, sem1)
            cc = pltpu.async_copy(rna.at[ridx_v], r_v, sem2)
            ca.wait()
            cb.wait()
            cc.wait()

            def row_body(r, c2):
                for c in range(DIM // LANES):
                    sl = pl.ds(c * LANES, LANES)
                    acc_v[r, sl] = acc_v[r, sl] + h_v[r, sl] + r_v[r, sl]
                return c2

            lax.fori_loop(0, CHUNK, row_body, 0)
            pltpu.sync_copy(acc_v, out.at[pl.ds(base, CHUNK)])
            return carry

        lax.fori_loop(0, n_chunks, chunk_body, 0)

    return emb_sum


def kernel(x_gene_id, x_connect_id, x_rna_type, basic_table, homo_table, rna_table):
    batch, seq = x_gene_id.shape
    n = batch * seq
    gid = x_gene_id.reshape(n).astype(jnp.int32)
    cid = x_connect_id.reshape(n).astype(jnp.int32)
    rid = x_rna_type.reshape(n).astype(jnp.int32)
    out = _build(n)(gid, cid, rid, basic_table, homo_table, rna_table)
    return out.reshape(batch, seq, DIM)


# speedup vs baseline: 2.8110x; 1.0302x over previous
import functools

import jax
import jax.numpy as jnp
from jax import lax
from jax.experimental import pallas as pl
from jax.experimental.pallas import tpu as pltpu
from jax.experimental.pallas import tpu_sc as plsc

DIM = 64
LANES = 16
CHUNK = 128


@functools.lru_cache(maxsize=None)
def _build(n_rows: int):
    info = plsc.get_sparse_core_info()
    num_workers = info.num_cores * info.num_subcores
    per_w = n_rows // num_workers
    n_chunks = per_w // CHUNK
    assert per_w * num_workers == n_rows and n_chunks * CHUNK == per_w
    assert n_chunks % 2 == 0
    half = n_chunks // 2

    mesh = plsc.VectorSubcoreMesh(core_axis_name="c", subcore_axis_name="s")

    @functools.partial(
        pl.kernel,
        mesh=mesh,
        compiler_params=pltpu.CompilerParams(use_tc_tiling_on_sc=False),
        out_type=jax.ShapeDtypeStruct((n_rows, DIM), jnp.float32),
        scratch_types=[
            pltpu.VMEM((n_chunks, CHUNK), jnp.int32),
            pltpu.VMEM((n_chunks, CHUNK), jnp.int32),
            pltpu.VMEM((n_chunks, CHUNK), jnp.int32),
            [pltpu.VMEM((CHUNK, DIM), jnp.float32) for _ in range(2)],
            [pltpu.VMEM((CHUNK, DIM), jnp.float32) for _ in range(2)],
            [pltpu.VMEM((CHUNK, DIM), jnp.float32) for _ in range(2)],
            pltpu.SemaphoreType.DMA,
            [pltpu.SemaphoreType.DMA for _ in range(2)],
            [pltpu.SemaphoreType.DMA for _ in range(2)],
        ],
    )
    def emb_sum(gid, cid, rid, basic, homo, rna, out,
                gidx_v, cidx_v, ridx_v, acc_v, h_v, r_v,
                sem_idx, sem_g, sem_st):
        w = lax.axis_index("s") * info.num_cores + lax.axis_index("c")
        wrow = w * n_chunks

        i0 = pltpu.async_copy(gid.at[pl.ds(wrow, n_chunks)], gidx_v, sem_idx)
        i1 = pltpu.async_copy(cid.at[pl.ds(wrow, n_chunks)], cidx_v, sem_idx)
        i2 = pltpu.async_copy(rid.at[pl.ds(wrow, n_chunks)], ridx_v, sem_idx)
        i0.wait()
        i1.wait()
        i2.wait()

        def fire_gathers(g, p):
            pltpu.async_copy(basic.at[gidx_v.at[g]], acc_v[p], sem_g[p])
            pltpu.async_copy(homo.at[cidx_v.at[g]], h_v[p], sem_g[p])
            pltpu.async_copy(rna.at[ridx_v.at[g]], r_v[p], sem_g[p])

        def wait_gathers(p):
            for _ in range(3):
                pltpu.make_async_copy(basic.at[gidx_v.at[0]], acc_v[p],
                                      sem_g[p]).wait()

        def fire_store(g, p):
            pltpu.async_copy(acc_v[p], out.at[pl.ds((wrow + g) * CHUNK, CHUNK)],
                             sem_st[p])

        def wait_store(p):
            pltpu.make_async_copy(acc_v[p],
                                  out.at[pl.ds(wrow * CHUNK, CHUNK)],
                                  sem_st[p]).wait()

        def compute(p):
            acc, h, r = acc_v[p], h_v[p], r_v[p]

            def row_body(rr, c2):
                for c in range(DIM // LANES):
                    sl = pl.ds(c * LANES, LANES)
                    plsc.addupdate(acc.at[rr, sl], h[rr, sl] + r[rr, sl])
                return c2

            lax.fori_loop(0, CHUNK, row_body, 0)

        fire_gathers(0, 0)

        def body(t, carry):
            a = 2 * t

            @pl.when(t > 0)
            def _():
                wait_store(1)

            fire_gathers(a + 1, 1)
            wait_gathers(0)
            compute(0)
            fire_store(a, 0)
            wait_store(0)

            @pl.when(t < half - 1)
            def _():
                fire_gathers(a + 2, 0)

            wait_gathers(1)
            compute(1)
            fire_store(a + 1, 1)
            return carry

        lax.fori_loop(0, half, body, 0)
        wait_store(1)

    return emb_sum


def kernel(x_gene_id, x_connect_id, x_rna_type, basic_table, homo_table, rna_table):
    batch, seq = x_gene_id.shape
    n = batch * seq
    gid = x_gene_id.reshape(n // CHUNK, CHUNK).astype(jnp.int32)
    cid = x_connect_id.reshape(n // CHUNK, CHUNK).astype(jnp.int32)
    rid = x_rna_type.reshape(n // CHUNK, CHUNK).astype(jnp.int32)
    out = _build(n)(gid, cid, rid, basic_table, homo_table, rna_table)
    return out.reshape(batch, seq, DIM)
